# baseline (speedup 1.0000x reference)
import functools

import jax
import jax.numpy as jnp
from jax import lax
from jax.experimental import pallas as pl
from jax.experimental.pallas import tpu as pltpu
from jax.experimental.pallas import tpu_sc as plsc

_BATCH = 4
_SEQ = 8192
_DIM = 1024
_NUM_WORKERS = 32
_ROWS_PER_W = _SEQ // _NUM_WORKERS
_CHUNK = 32
_NBUF = 3


def _sc_broadcast(table):
    mesh = plsc.VectorSubcoreMesh(core_axis_name="c", subcore_axis_name="s")
    n = _ROWS_PER_W // _CHUNK

    @functools.partial(
        pl.kernel,
        mesh=mesh,
        out_type=jax.ShapeDtypeStruct((_BATCH, _SEQ, _DIM), jnp.float32),
        scratch_types=(
            [pltpu.VMEM((_CHUNK, _DIM), jnp.float32) for _ in range(_NBUF)]
            + [pltpu.SemaphoreType.DMA for _ in range(2 * _NBUF)]
        ),
    )
    def k(table_hbm, out_hbm, *refs):
        bufs = refs[:_NBUF]
        rsems = refs[_NBUF : 2 * _NBUF]
        wsems = refs[2 * _NBUF :]
        wid = lax.axis_index("s") * 2 + lax.axis_index("c")
        base = wid * _ROWS_PER_W

        rd = [None] * n
        wr = [[] for _ in range(n)]

        def issue_read(j):
            row = base + j * _CHUNK
            rd[j] = pltpu.async_copy(
                table_hbm.at[pl.ds(row, _CHUNK)], bufs[j % _NBUF], rsems[j % _NBUF]
            )

        for j in range(min(_NBUF, n)):
            issue_read(j)
        for i in range(n):
            b = i % _NBUF
            row = base + i * _CHUNK
            rd[i].wait()
            for bb in range(_BATCH):
                wr[i].append(
                    pltpu.async_copy(
                        bufs[b], out_hbm.at[bb, pl.ds(row, _CHUNK)], wsems[b]
                    )
                )
            nxt = i + 1
            if _NBUF <= nxt < n:
                for h in wr[nxt - _NBUF]:
                    h.wait()
                issue_read(nxt)
        for i in range(max(0, n - _NBUF), n):
            for h in wr[i]:
                h.wait()

    return k(table)


def kernel(x, table):
    del x
    return _sc_broadcast(table)
